# Initial kernel scaffold; baseline (speedup 1.0000x reference)
import jax
import jax.numpy as jnp
from jax.experimental import pallas as pl


def kernel(logits, neighborhood_temps):
    raise NotImplementedError("write your pallas kernel here")


# speedup vs baseline: 198.2063x; 198.2063x over previous
import functools

import jax
import jax.numpy as jnp
from jax.experimental import pallas as pl
from jax.experimental.pallas import tpu as pltpu

_B, _C, _H, _W = 8, 19, 512, 512
_NEIGH_W = 3
_EPS = 1e-12
_HB = 128
_HALO = 8


def _argmax_c(x):
    m = x[0]
    idx = jnp.zeros(x.shape[1:], dtype=jnp.int32)
    for c in range(1, x.shape[0]):
        pred = x[c] > m
        m = jnp.where(pred, x[c], m)
        idx = jnp.where(pred, c, idx)
    return idx


def _nectar_kernel(inv_table_ref, logits_ref, top_ref, bot_ref, out_ref):
    i = pl.program_id(1)
    n_i = pl.num_programs(1)

    x = logits_ref[0]
    lab = _argmax_c(x)

    lab_top = _argmax_c(top_ref[0, :, _HALO - 1 : _HALO, :])
    lab_bot = _argmax_c(bot_ref[0, :, 0:1, :])
    minus1 = jnp.full_like(lab_top, -1)
    lab_top = jnp.where(i == 0, minus1, lab_top)
    lab_bot = jnp.where(i == n_i - 1, minus1, lab_bot)

    L = jnp.concatenate([lab_top, lab, lab_bot], axis=0)

    count = jnp.zeros(lab.shape, dtype=jnp.int32)
    mcol = jnp.full((_HB, 1), -1, dtype=jnp.int32)
    for di in range(3):
        rows = L[di : di + _HB, :]
        for dj in range(3):
            if dj == 0:
                s = jnp.concatenate([mcol, rows[:, : _W - 1]], axis=1)
            elif dj == 2:
                s = jnp.concatenate([rows[:, 1:], mcol], axis=1)
            else:
                s = rows
            count = count + (s == lab).astype(jnp.int32)
    n = count - 1

    inv = jnp.full(lab.shape, inv_table_ref[0], dtype=jnp.float32)
    for k in range(1, _NEIGH_W * _NEIGH_W):
        inv = jnp.where(n == k, inv_table_ref[k], inv)

    out_ref[0] = x * inv[None, :, :]


@jax.jit
def kernel(logits, neighborhood_temps):
    inv_table = 1.0 / (jax.nn.relu(neighborhood_temps) + _EPS)
    n_i = _H // _HB
    hb8 = _HB // _HALO

    grid = (_B, n_i)
    in_specs = [
        pl.BlockSpec((1, _C, _HB, _W), lambda b, i, *_: (b, 0, i, 0)),
        pl.BlockSpec(
            (1, _C, _HALO, _W),
            lambda b, i, *_: (b, 0, jnp.maximum(i * hb8 - 1, 0), 0),
        ),
        pl.BlockSpec(
            (1, _C, _HALO, _W),
            lambda b, i, *_: (b, 0, jnp.minimum((i + 1) * hb8, _H // _HALO - 1), 0),
        ),
    ]
    out_spec = pl.BlockSpec((1, _C, _HB, _W), lambda b, i, *_: (b, 0, i, 0))

    return pl.pallas_call(
        _nectar_kernel,
        grid_spec=pltpu.PrefetchScalarGridSpec(
            num_scalar_prefetch=1,
            grid=grid,
            in_specs=in_specs,
            out_specs=out_spec,
        ),
        out_shape=jax.ShapeDtypeStruct(logits.shape, logits.dtype),
    )(inv_table, logits, logits, logits)


# speedup vs baseline: 205.1075x; 1.0348x over previous
import functools

import jax
import jax.numpy as jnp
from jax.experimental import pallas as pl
from jax.experimental.pallas import tpu as pltpu

_B, _C, _H, _W = 8, 19, 512, 512
_NEIGH_W = 3
_EPS = 1e-12
_HB = 256
_HALO = 8


def _argmax_c(x):
    m = x[0]
    idx = jnp.zeros(x.shape[1:], dtype=jnp.int32)
    for c in range(1, x.shape[0]):
        pred = x[c] > m
        m = jnp.where(pred, x[c], m)
        idx = jnp.where(pred, c, idx)
    return idx


def _nectar_kernel(inv_table_ref, logits_ref, top_ref, bot_ref, out_ref):
    i = pl.program_id(1)
    n_i = pl.num_programs(1)

    x = logits_ref[0]
    lab = _argmax_c(x)

    lab_top = _argmax_c(top_ref[0, :, _HALO - 1 : _HALO, :])
    lab_bot = _argmax_c(bot_ref[0, :, 0:1, :])
    minus1 = jnp.full_like(lab_top, -1)
    lab_top = jnp.where(i == 0, minus1, lab_top)
    lab_bot = jnp.where(i == n_i - 1, minus1, lab_bot)

    L = jnp.concatenate([lab_top, lab, lab_bot], axis=0)

    count = jnp.zeros(lab.shape, dtype=jnp.int32)
    mcol = jnp.full((_HB, 1), -1, dtype=jnp.int32)
    for di in range(3):
        rows = L[di : di + _HB, :]
        for dj in range(3):
            if dj == 0:
                s = jnp.concatenate([mcol, rows[:, : _W - 1]], axis=1)
            elif dj == 2:
                s = jnp.concatenate([rows[:, 1:], mcol], axis=1)
            else:
                s = rows
            count = count + (s == lab).astype(jnp.int32)
    n = count - 1

    inv = jnp.full(lab.shape, inv_table_ref[0], dtype=jnp.float32)
    for k in range(1, _NEIGH_W * _NEIGH_W):
        inv = jnp.where(n == k, inv_table_ref[k], inv)

    out_ref[0] = x * inv[None, :, :]


@jax.jit
def kernel(logits, neighborhood_temps):
    inv_table = 1.0 / (jax.nn.relu(neighborhood_temps) + _EPS)
    n_i = _H // _HB
    hb8 = _HB // _HALO

    grid = (_B, n_i)
    in_specs = [
        pl.BlockSpec((1, _C, _HB, _W), lambda b, i, *_: (b, 0, i, 0)),
        pl.BlockSpec(
            (1, _C, _HALO, _W),
            lambda b, i, *_: (b, 0, jnp.maximum(i * hb8 - 1, 0), 0),
        ),
        pl.BlockSpec(
            (1, _C, _HALO, _W),
            lambda b, i, *_: (b, 0, jnp.minimum((i + 1) * hb8, _H // _HALO - 1), 0),
        ),
    ]
    out_spec = pl.BlockSpec((1, _C, _HB, _W), lambda b, i, *_: (b, 0, i, 0))

    return pl.pallas_call(
        _nectar_kernel,
        grid_spec=pltpu.PrefetchScalarGridSpec(
            num_scalar_prefetch=1,
            grid=grid,
            in_specs=in_specs,
            out_specs=out_spec,
        ),
        out_shape=jax.ShapeDtypeStruct(logits.shape, logits.dtype),
    )(inv_table, logits, logits, logits)


# speedup vs baseline: 207.7092x; 1.0127x over previous
import jax
import jax.numpy as jnp
from jax.experimental import pallas as pl
from jax.experimental.pallas import tpu as pltpu

_B, _C, _H, _W = 8, 19, 512, 512
_NEIGH_W = 3
_EPS = 1e-12
_HB = 256
_HALO = 8


def _argmax_c(x):
    m = x[0]
    idx = jnp.zeros(x.shape[1:], dtype=jnp.int32)
    for c in range(1, x.shape[0]):
        pred = x[c] > m
        m = jnp.where(pred, x[c], m)
        idx = jnp.where(pred, c, idx)
    return idx


def _nectar_kernel(inv_table_ref, logits_ref, bot_ref, out_ref, carry_ref):
    i = pl.program_id(1)
    n_i = pl.num_programs(1)

    x = logits_ref[0]
    lab = _argmax_c(x)

    minus1 = jnp.full((1, _W), -1, dtype=jnp.int32)
    lab_top = jnp.where(i == 0, minus1, carry_ref[0:1, :])
    lab_bot = _argmax_c(bot_ref[0, :, 0:1, :])
    lab_bot = jnp.where(i == n_i - 1, minus1, lab_bot)

    L = jnp.concatenate([lab_top, lab, lab_bot], axis=0)
    carry_ref[0:1, :] = lab[_HB - 1 : _HB, :]

    count = jnp.zeros(lab.shape, dtype=jnp.int32)
    mcol = jnp.full((_HB, 1), -1, dtype=jnp.int32)
    for di in range(3):
        rows = L[di : di + _HB, :]
        for dj in range(3):
            if dj == 0:
                s = jnp.concatenate([mcol, rows[:, : _W - 1]], axis=1)
            elif dj == 2:
                s = jnp.concatenate([rows[:, 1:], mcol], axis=1)
            else:
                s = rows
            count = count + (s == lab).astype(jnp.int32)
    n = count - 1

    inv = jnp.full(lab.shape, inv_table_ref[0], dtype=jnp.float32)
    for k in range(1, _NEIGH_W * _NEIGH_W):
        inv = jnp.where(n == k, inv_table_ref[k], inv)

    out_ref[0] = x * inv[None, :, :]


@jax.jit
def kernel(logits, neighborhood_temps):
    inv_table = 1.0 / (jax.nn.relu(neighborhood_temps) + _EPS)
    n_i = _H // _HB

    grid = (_B, n_i)
    in_specs = [
        pl.BlockSpec((1, _C, _HB, _W), lambda b, i, *_: (b, 0, i, 0)),
        pl.BlockSpec(
            (1, _C, _HALO, _W),
            lambda b, i, *_: (
                b,
                0,
                jnp.minimum((i + 1) * (_HB // _HALO), _H // _HALO - 1),
                0,
            ),
        ),
    ]
    out_spec = pl.BlockSpec((1, _C, _HB, _W), lambda b, i, *_: (b, 0, i, 0))

    return pl.pallas_call(
        _nectar_kernel,
        grid_spec=pltpu.PrefetchScalarGridSpec(
            num_scalar_prefetch=1,
            grid=grid,
            in_specs=in_specs,
            out_specs=out_spec,
            scratch_shapes=[pltpu.VMEM((8, _W), jnp.int32)],
        ),
        out_shape=jax.ShapeDtypeStruct(logits.shape, logits.dtype),
    )(inv_table, logits, logits)
